# baseline (speedup 1.0000x reference)
import jax
import jax.numpy as jnp
from jax import lax
from jax.experimental import pallas as pl
from jax.experimental.pallas import tpu as pltpu
from jax.experimental.pallas import tpu_sc as plsc

_NC, _NS = 2, 16
_NW = _NC * _NS
_CHUNK = 800


def _sc_lookup(table, idx, n, v, d):
    bpw = n // _NW
    nchunk = bpw // _CHUNK
    mesh = plsc.VectorSubcoreMesh(
        core_axis_name="core", subcore_axis_name="subcore"
    )

    @pl.kernel(
        out_type=jax.ShapeDtypeStruct((n, d), jnp.float32),
        mesh=mesh,
        compiler_params=pltpu.CompilerParams(
            use_tc_tiling_on_sc=False, needs_layout_passes=False
        ),
        scratch_types=[
            pltpu.VMEM_SHARED((v, d), jnp.float32),
            pltpu.VMEM((bpw,), jnp.int32),
            pltpu.VMEM((_CHUNK, d), jnp.float32),
            pltpu.VMEM((_CHUNK, d), jnp.float32),
            pltpu.SemaphoreType.DMA,
            pltpu.SemaphoreType.DMA,
            pltpu.SemaphoreType.DMA,
            pltpu.SemaphoreType.DMA,
        ],
    )
    def k(t_hbm, i_hbm, o_hbm, tab_sh, idx_v, rows0, rows1, gs0, gs1, os0, os1):
        sid = lax.axis_index("subcore")
        wid = sid * _NC + lax.axis_index("core")
        base = wid * bpw

        @pl.when(sid == 0)
        def _():
            pltpu.sync_copy(t_hbm, tab_sh)

        pltpu.sync_copy(i_hbm.at[pl.ds(base, bpw)], idx_v)
        plsc.subcore_barrier()

        rows = (rows0, rows1)
        gsems = (gs0, gs1)
        osems = (os0, os1)

        def phase(kc, b):
            @pl.when(kc >= 2)
            def _():
                pltpu.make_async_copy(
                    rows[b], o_hbm.at[pl.ds(0, _CHUNK)], osems[b]
                ).wait()

            pltpu.async_copy(
                tab_sh.at[idx_v.at[pl.ds(kc * _CHUNK, _CHUNK)]],
                rows[b],
                gsems[b],
            ).wait()
            pltpu.async_copy(
                rows[b], o_hbm.at[pl.ds(base + kc * _CHUNK, _CHUNK)], osems[b]
            )

        @pl.loop(0, nchunk, step=2)
        def _(kc):
            phase(kc, 0)
            phase(kc + 1, 1)

        pltpu.make_async_copy(rows0, o_hbm.at[pl.ds(0, _CHUNK)], os0).wait()
        pltpu.make_async_copy(rows1, o_hbm.at[pl.ds(0, _CHUNK)], os1).wait()

    return k(table, idx)


def kernel(x, table):
    b, s = x.shape
    v, d = table.shape
    n = b * s
    out = _sc_lookup(table, x.reshape(n), n, v, d)
    return out.reshape(b, s, d)


# speedup vs baseline: 2.2516x; 2.2516x over previous
import jax
import jax.numpy as jnp
from jax import lax
from jax.experimental import pallas as pl
from jax.experimental.pallas import tpu as pltpu
from jax.experimental.pallas import tpu_sc as plsc

_NC, _NS = 2, 16
_NW = _NC * _NS
_L = 16
_CHUNK = 800


def _sc_lookup(table, idx, n, v, d):
    bpw = n // _NW
    nchunk = bpw // _CHUNK
    mesh = plsc.VectorSubcoreMesh(
        core_axis_name="core", subcore_axis_name="subcore"
    )

    @pl.kernel(
        out_type=jax.ShapeDtypeStruct((n * d,), jnp.float32),
        mesh=mesh,
        compiler_params=pltpu.CompilerParams(
            use_tc_tiling_on_sc=False, needs_layout_passes=False
        ),
        scratch_types=[
            pltpu.VMEM((v * d,), jnp.float32),
            pltpu.VMEM((bpw,), jnp.int32),
            pltpu.VMEM((_CHUNK * d,), jnp.float32),
            pltpu.VMEM((_CHUNK * d,), jnp.float32),
            pltpu.SemaphoreType.DMA,
            pltpu.SemaphoreType.DMA,
        ],
    )
    def k(t_hbm, i_hbm, o_hbm, tab_v, idx_v, rows0, rows1, sem0, sem1):
        wid = lax.axis_index("subcore") * _NC + lax.axis_index("core")
        base = wid * bpw
        pltpu.sync_copy(t_hbm, tab_v)
        pltpu.sync_copy(i_hbm.at[pl.ds(base, bpw)], idx_v)

        rows = (rows0, rows1)
        sems = (sem0, sem1)

        def do_chunk(kc, b):
            rb, sb = rows[b], sems[b]

            @pl.when(kc >= 2)
            def _():
                pltpu.make_async_copy(
                    rb, o_hbm.at[pl.ds(0, _CHUNK * d)], sb
                ).wait()

            co = kc * _CHUNK

            @plsc.parallel_loop(0, _CHUNK // _L, unroll=2)
            def _(g):
                off = g * _L
                idxv = idx_v[pl.ds(co + off, _L)] * d
                rbase = off * d
                for jj in range(_L):
                    a = idxv[jj]
                    r = rbase + jj * d
                    rb[pl.ds(r, _L)] = tab_v[pl.ds(a, _L)]
                    rb[pl.ds(r + _L, _L)] = tab_v[pl.ds(a + _L, _L)]

            pltpu.async_copy(
                rb, o_hbm.at[pl.ds((base + kc * _CHUNK) * d, _CHUNK * d)], sb
            )

        @pl.loop(0, nchunk, step=2)
        def _(kc):
            do_chunk(kc, 0)
            do_chunk(kc + 1, 1)

        pltpu.make_async_copy(rows0, o_hbm.at[pl.ds(0, _CHUNK * d)], sem0).wait()
        pltpu.make_async_copy(rows1, o_hbm.at[pl.ds(0, _CHUNK * d)], sem1).wait()

    return k(table, idx)


def kernel(x, table):
    b, s = x.shape
    v, d = table.shape
    n = b * s
    out = _sc_lookup(table.reshape(v * d), x.reshape(n), n, v, d)
    return out.reshape(b, s, d)
